# baseline (speedup 1.0000x reference)
import functools

import jax
import jax.numpy as jnp
from jax import lax
from jax.experimental import pallas as pl
from jax.experimental.pallas import tpu as pltpu
from jax.experimental.pallas import tpu_sc as plsc

N_ROWS = 1024 * 50
C = 1000
NC, NS = 2, 16
NW = NC * NS
RPW = N_ROWS // NW
L = 16
ZW = 100_000
NZDMA = RPW * C // ZW
SCW = 64
NSC = RPW // SCW


def _one_hot_sc(x_flat):
    mesh = plsc.VectorSubcoreMesh(core_axis_name="c", subcore_axis_name="s")

    @functools.partial(
        pl.kernel,
        out_type=jax.ShapeDtypeStruct((N_ROWS * C,), jnp.float32),
        mesh=mesh,
        scratch_types=[
            pltpu.VMEM((RPW,), jnp.int32),
            pltpu.VMEM((ZW,), jnp.float32),
            pltpu.VMEM((NSC, SCW), jnp.int32),
            pltpu.VMEM((SCW,), jnp.float32),
            pltpu.SemaphoreType.DMA,
            pltpu.SemaphoreType.DMA,
        ],
    )
    def body(x_hbm, out_hbm, xv, zbuf, pos, ones, zsem, ssem):
        wid = lax.axis_index("s") * NC + lax.axis_index("c")
        base = pl.multiple_of(wid * RPW, RPW)

        zeros_f = jnp.zeros((L,), jnp.float32)

        def zero_body(i, _):
            zbuf[pl.ds(pl.multiple_of(i * L, L), L)] = zeros_f
            return _
        lax.fori_loop(0, ZW // L, zero_body, None, unroll=8)

        for t in range(NZDMA):
            pltpu.async_copy(
                zbuf, out_hbm.at[pl.ds(base * C + t * ZW, ZW)], zsem)

        pltpu.sync_copy(x_hbm.at[pl.ds(base, RPW)], xv)
        lanes = lax.iota(jnp.int32, L)
        for c in range(SCW // L):
            ones[pl.ds(c * L, L)] = jnp.ones((L,), jnp.float32)

        def pos_body(j, _):
            row0 = base + j * SCW
            for c in range(SCW // L):
                idx16 = xv[pl.ds(j * SCW + c * L, L)]
                pos[j, pl.ds(c * L, L)] = (row0 + c * L + lanes) * C + idx16
            return _
        lax.fori_loop(0, NSC, pos_body, None)

        for t in range(NZDMA):
            pltpu.make_async_copy(
                zbuf, out_hbm.at[pl.ds(0, ZW)], zsem).wait()
        for j in range(NSC):
            pltpu.async_copy(ones, out_hbm.at[pos.at[j]], ssem)
        for j in range(NSC):
            pltpu.make_async_copy(
                ones, out_hbm.at[pos.at[0]], ssem).wait()

    return body(x_flat)


@jax.jit
def kernel(x):
    out = _one_hot_sc(x.reshape(-1))
    return out.reshape(x.shape[0], x.shape[1], C)
